# Initial kernel scaffold; baseline (speedup 1.0000x reference)
import jax
import jax.numpy as jnp
from jax.experimental import pallas as pl


def kernel(x, edge_index, W1, b1, W2, b2):
    raise NotImplementedError("write your pallas kernel here")


# speedup vs baseline: 10.1789x; 10.1789x over previous
import functools

import jax
import jax.numpy as jnp
from jax import lax
from jax.experimental import pallas as pl
from jax.experimental.pallas import tpu as pltpu
from jax.experimental.pallas import tpu_sc as plsc

N = 10000
D = 128
E = 320000

NC = 2
NS = 16
NW = NC * NS

CH = 128
K = -(-E // (NW * CH))
EW = K * CH
E_PAD = NW * EW
PAD = E_PAD - E

NROW = 12800
RPT = NROW // NS
DUMP = N
ZR = 80

R = 400
GRID = N // R
POFF = NROW // R


def _sc_deg_body(dst_hbm, out_hbm, dstv, ones, zb, acc):
    c = lax.axis_index("c")
    s = lax.axis_index("s")
    wid = c * NS + s

    def fill(i, _):
        ones[i // 8, pl.ds((i % 8) * 16, 16)] = jnp.ones((16,), jnp.float32)
        zb[i % (ZR * 8) // 8, pl.ds(i % 8 * 16, 16)] = jnp.zeros((16,), jnp.float32)
        return 0

    lax.fori_loop(0, CH * 8, fill, 0)

    def zero(t, _):
        pltpu.sync_copy(zb, acc.at[pl.ds(s * RPT + t * ZR, ZR)])
        return 0

    lax.fori_loop(0, RPT // ZR, zero, 0)
    plsc.subcore_barrier()

    def chunk(j, _):
        pltpu.sync_copy(dst_hbm.at[pl.ds(wid * EW + j * CH, CH)], dstv)
        pltpu.sync_copy(ones, acc.at[dstv], add=True)
        return 0

    lax.fori_loop(0, K, chunk, 0)
    plsc.subcore_barrier()
    pltpu.sync_copy(
        acc.at[pl.ds(s * RPT, RPT)],
        out_hbm.at[pl.ds(c * NROW + s * RPT, RPT)],
    )


def _sc_agg_body(g_hbm, src_hbm, dst_hbm, out_hbm, srcv, dstv, rows, zb, acc, sem):
    c = lax.axis_index("c")
    s = lax.axis_index("s")
    wid = c * NS + s

    def fill(i, _):
        zb[i // 8, pl.ds((i % 8) * 16, 16)] = jnp.zeros((16,), jnp.float32)
        return 0

    lax.fori_loop(0, ZR * 8, fill, 0)

    def zero(t, _):
        pltpu.sync_copy(zb, acc.at[pl.ds(s * RPT + t * ZR, ZR)])
        return 0

    lax.fori_loop(0, RPT // ZR, zero, 0)
    plsc.subcore_barrier()

    def chunk(j, _):
        base = wid * EW + j * CH
        pltpu.sync_copy(src_hbm.at[pl.ds(base, CH)], srcv)
        pltpu.sync_copy(dst_hbm.at[pl.ds(base, CH)], dstv)
        pltpu.async_copy(g_hbm.at[srcv], rows, sem).wait()
        pltpu.sync_copy(rows, acc.at[dstv], add=True)
        return 0

    lax.fori_loop(0, K, chunk, 0)
    plsc.subcore_barrier()
    pltpu.sync_copy(
        acc.at[pl.ds(s * RPT, RPT)],
        out_hbm.at[pl.ds(c * NROW + s * RPT, RPT)],
    )


@functools.cache
def _sc_kernels():
    mesh = plsc.VectorSubcoreMesh(
        core_axis_name="c", subcore_axis_name="s", num_cores=NC, num_subcores=NS
    )
    sc_deg = pl.kernel(
        _sc_deg_body,
        out_type=jax.ShapeDtypeStruct((NC * NROW, D), jnp.float32),
        mesh=mesh,
        scratch_types=[
            pltpu.VMEM((CH,), jnp.int32),
            pltpu.VMEM((CH, D), jnp.float32),
            pltpu.VMEM((ZR, D), jnp.float32),
            pltpu.VMEM_SHARED((NROW, D), jnp.float32),
        ],
    )
    sc_agg = pl.kernel(
        _sc_agg_body,
        out_type=jax.ShapeDtypeStruct((NC * NROW, D), jnp.float32),
        mesh=mesh,
        scratch_types=[
            pltpu.VMEM((CH,), jnp.int32),
            pltpu.VMEM((CH,), jnp.int32),
            pltpu.VMEM((CH, D), jnp.float32),
            pltpu.VMEM((ZR, D), jnp.float32),
            pltpu.VMEM_SHARED((NROW, D), jnp.float32),
            pltpu.SemaphoreType.DMA,
        ],
    )
    return sc_deg, sc_agg


def _dinv(da_ref, db_ref):
    return lax.rsqrt(1.0 + da_ref[:, 0:1] + db_ref[:, 0:1])


def _mm1_body(x_ref, w_ref, da_ref, db_ref, o_ref):
    h = jnp.dot(x_ref[...], w_ref[...], preferred_element_type=jnp.float32)
    o_ref[...] = _dinv(da_ref, db_ref) * h


def _mm2_body(p0_ref, p1_ref, g_ref, da_ref, db_ref, b_ref, w_ref, o_ref):
    dinv = _dinv(da_ref, db_ref)
    t = dinv * (p0_ref[...] + p1_ref[...] + g_ref[...]) + b_ref[...]
    h = jnp.dot(t, w_ref[...], preferred_element_type=jnp.float32)
    o_ref[...] = dinv * h


def _fin_body(p0_ref, p1_ref, g_ref, da_ref, db_ref, b_ref, o_ref):
    dinv = _dinv(da_ref, db_ref)
    o_ref[...] = dinv * (p0_ref[...] + p1_ref[...] + g_ref[...]) + b_ref[...]


_row = pl.BlockSpec((R, D), lambda i: (i, 0))
_row1 = pl.BlockSpec((R, D), lambda i: (i + POFF, 0))
_deg0 = pl.BlockSpec((R, D), lambda i: (i, 0))
_deg1 = pl.BlockSpec((R, D), lambda i: (i + POFF, 0))
_wspec = pl.BlockSpec((D, D), lambda i: (0, 0))
_bspec = pl.BlockSpec((1, D), lambda i: (0, 0))
_oshape = jax.ShapeDtypeStruct((N, D), jnp.float32)

_mm1 = pl.pallas_call(
    _mm1_body, grid=(GRID,),
    in_specs=[_row, _wspec, _deg0, _deg1],
    out_specs=_row, out_shape=_oshape,
)
_mm2 = pl.pallas_call(
    _mm2_body, grid=(GRID,),
    in_specs=[_row, _row1, _row, _deg0, _deg1, _bspec, _wspec],
    out_specs=_row, out_shape=_oshape,
)
_fin = pl.pallas_call(
    _fin_body, grid=(GRID,),
    in_specs=[_row, _row1, _row, _deg0, _deg1, _bspec],
    out_specs=_row, out_shape=_oshape,
)


def kernel(x, edge_index, W1, b1, W2, b2):
    src = edge_index[0]
    dst = edge_index[1]
    srcp = jnp.concatenate([src, jnp.zeros((PAD,), jnp.int32)])
    dstp = jnp.concatenate([dst, jnp.full((PAD,), DUMP, jnp.int32)])

    sc_deg, sc_agg = _sc_kernels()
    degp = sc_deg(dstp)
    g1 = _mm1(x, W1, degp, degp)
    s1 = sc_agg(g1, srcp, dstp)
    g2 = _mm2(s1, s1, g1, degp, degp, b1.reshape(1, D), W2)
    s2 = sc_agg(g2, srcp, dstp)
    return _fin(s2, s2, g2, degp, degp, b2.reshape(1, D))
